# baseline (speedup 1.0000x reference)
import jax
import jax.numpy as jnp
from jax import lax
from jax.experimental import pallas as pl
from jax.experimental.pallas import tpu as pltpu
from jax.experimental.pallas import tpu_sc as plsc

Z = 0.8
NC, NS = 2, 16
NW = NC * NS
K = 80
SJ = 10
R = 400
D = 128


def _pre_body(x_ref, m_ref, w0t_ref, b0_ref, w1t_ref, b1_ref, out_ref):
    x = x_ref[...]
    x0 = jnp.maximum(
        jnp.dot(x, w0t_ref[...], preferred_element_type=jnp.float32)
        + b0_ref[...], 0.0)
    x1 = jnp.maximum(
        jnp.dot(x, w1t_ref[...], preferred_element_type=jnp.float32)
        + b1_ref[...], 0.0)
    m = m_ref[...]
    a = Z * x1 + (1.0 - Z) * x0
    b = Z * x0 + (1.0 - Z) * x1
    out_ref[...] = b + m * (a - b)


def _sc_body(ei5_hbm, w5_hbm, xb_hbm, zx_hbm, px_hbm,
             src_sb, dst_sb, w_sb, r0, r1, r2,
             g0, g1, g2, s0, s1, s2, isem, acc_x):
    rows = (r0, r1, r2)
    gsem = (g0, g1, g2)
    ssem = (s0, s1, s2)
    NB = 3
    npad = acc_x.shape[0]
    nsuper = ei5_hbm.shape[2]
    nchunk = nsuper * SJ
    c = lax.axis_index("c")
    s = lax.axis_index("s")
    wid = c * NS + s
    rpt = npad // NS
    lane0 = jnp.where(lax.iota(jnp.int32, 16) == 0, 1.0, 0.0)
    gdn = lax.GatherDimensionNumbers(
        offset_dims=(), collapsed_slice_dims=(0,), start_index_map=(0,))

    pltpu.sync_copy(zx_hbm, acc_x.at[pl.ds(s * rpt, rpt)])
    plsc.subcore_barrier()

    def super_load_async(p, slot):
        pltpu.async_copy(ei5_hbm.at[0, wid, p], src_sb.at[slot], isem)
        pltpu.async_copy(ei5_hbm.at[1, wid, p], dst_sb.at[slot], isem)
        pltpu.async_copy(w5_hbm.at[wid, p], w_sb.at[slot], isem)

    def super_wait():
        pltpu.make_async_copy(ei5_hbm.at[0, wid, 0], src_sb.at[0], isem).wait()
        pltpu.make_async_copy(ei5_hbm.at[1, wid, 0], dst_sb.at[0], isem).wait()
        pltpu.make_async_copy(w5_hbm.at[wid, 0], w_sb.at[0], isem).wait()

    def scale_chunk(pq, jj, rb):
        @plsc.parallel_loop(0, K // 16, 1, unroll=K // 16)
        def escale(g):
            wg = w_sb[pq, jj, pl.ds(g * 16, 16)]
            for l in range(16):
                idx = (jnp.zeros((16,), jnp.int32) + l).reshape(16, 1)
                wb = lax.gather(wg, idx, gdn, (1,),
                                mode=lax.GatherScatterMode.PROMISE_IN_BOUNDS)
                e = g * 16 + l
                for d in range(D // 16):
                    rb[e, pl.ds(d * 16, 16)] = rb[e, pl.ds(d * 16, 16)] * wb

    def iter_one(j, b, wait_pred, gather_j2, gather_pred=None):
        bp = (b + 2) % NB
        pq = (j // SJ) % 2
        jj = j % SJ
        pltpu.make_async_copy(xb_hbm.at[dst_sb.at[pq, jj]], rows[b],
                              gsem[b]).wait()
        scale_chunk(pq, jj, rows[b])

        def retire():
            pltpu.make_async_copy(rows[bp], acc_x.at[src_sb.at[pq, jj]],
                                  ssem[bp]).wait()
        if wait_pred is None:
            retire()
        else:
            pl.when(wait_pred)(retire)

        if gather_j2 is not None:
            p = j // SJ
            pl.when((jj == 1) & (p + 1 < nsuper))(
                lambda: super_load_async(p + 1, (p + 1) % 2))
            pl.when((jj == SJ - 2) & (gather_j2 < nchunk))(super_wait)

            def launch():
                gpq = (gather_j2 // SJ) % 2
                gjj = gather_j2 % SJ
                pltpu.async_copy(xb_hbm.at[dst_sb.at[gpq, gjj]], rows[bp],
                                 gsem[bp])
            if gather_pred is None:
                launch()
            else:
                pl.when(gather_pred)(launch)
        pltpu.async_copy(rows[b], acc_x.at[src_sb.at[pq, jj]], ssem[b],
                         add=True)

    pltpu.sync_copy(ei5_hbm.at[0, wid, 0], src_sb.at[0])
    pltpu.sync_copy(ei5_hbm.at[1, wid, 0], dst_sb.at[0])
    pltpu.sync_copy(w5_hbm.at[wid, 0], w_sb.at[0])
    super_load_async(1, 1)
    pltpu.async_copy(xb_hbm.at[dst_sb.at[0, 0]], rows[0], gsem[0])
    pltpu.async_copy(xb_hbm.at[dst_sb.at[0, 1]], rows[1], gsem[1])

    nloop = nchunk // 3

    def body(i, carry):
        j0 = 3 * i
        iter_one(j0, 0, i > 0, j0 + 2, j0 + 2 < nchunk)
        iter_one(j0 + 1, 1, None, j0 + 3, j0 + 3 < nchunk)
        iter_one(j0 + 2, 2, None, j0 + 4, j0 + 4 < nchunk)
        return carry
    lax.fori_loop(0, nloop, body, 0)
    for je in range(3 * nloop, nchunk):
        iter_one(je, je % 3, None, None)
    blast = (nchunk - 1) % 3
    lq, lj = ((nchunk - 1) // SJ) % 2, (nchunk - 1) % SJ
    pltpu.make_async_copy(rows[blast], acc_x.at[src_sb.at[lq, lj]],
                          ssem[blast]).wait()

    plsc.subcore_barrier()
    pltpu.sync_copy(acc_x.at[pl.ds(s * rpt, rpt)],
                    px_hbm.at[c, pl.ds(s * rpt, rpt)])


def _deg_body(ei3_hbm, w3_hbm, zw_hbm, pw_hbm, src_a, w_a, wrow_v, acc_w):
    npad = acc_w.shape[0]
    nchkd = src_a.shape[0]
    kd = src_a.shape[1]
    c = lax.axis_index("c")
    s = lax.axis_index("s")
    wid = c * NS + s
    rpt = npad // NS
    lane0 = jnp.where(lax.iota(jnp.int32, 16) == 0, 1.0, 0.0)
    gdn = lax.GatherDimensionNumbers(
        offset_dims=(), collapsed_slice_dims=(0,), start_index_map=(0,))

    pltpu.sync_copy(zw_hbm, acc_w.at[pl.ds(s * rpt, rpt)])
    pltpu.sync_copy(ei3_hbm.at[0, wid], src_a)
    pltpu.sync_copy(w3_hbm.at[wid], w_a)
    plsc.subcore_barrier()

    def chunk(j, carry):
        @plsc.parallel_loop(0, kd // 16, 1, unroll=8)
        def build(g):
            wg = w_a[j, pl.ds(g * 16, 16)]
            for l in range(16):
                idx = (jnp.zeros((16,), jnp.int32) + l).reshape(16, 1)
                wb = lax.gather(wg, idx, gdn, (1,),
                                mode=lax.GatherScatterMode.PROMISE_IN_BOUNDS)
                wrow_v[g * 16 + l, pl.ds(0, 16)] = wb * lane0
        pltpu.sync_copy(wrow_v, acc_w.at[src_a.at[j]], add=True)
        return carry
    lax.fori_loop(0, nchkd, chunk, 0)

    plsc.subcore_barrier()
    pltpu.sync_copy(acc_w.at[pl.ds(s * rpt, rpt)],
                    pw_hbm.at[c, pl.ds(s * rpt, rpt)])


def _post_body(px_ref, pw_ref, x_ref, m_ref,
               a0_ref, c0_ref, a1_ref, c1_ref,
               bias0_ref, bias1_ref, g_ref, be_ref, out_ref):
    accx = px_ref[0] + px_ref[1]
    deg = pw_ref[0, :, 0:1] + pw_ref[1, :, 0:1]
    deg = jnp.where(deg < 0.5, deg + 1.0, deg)
    xm = accx / deg
    mu = jnp.mean(xm, axis=1, keepdims=True)
    var = jnp.mean((xm - mu) * (xm - mu), axis=1, keepdims=True)
    xn = (xm - mu) * lax.rsqrt(var + 1e-5) * g_ref[...] + be_ref[...]
    xin = x_ref[...]
    y0 = (jnp.dot(xn, a0_ref[...], preferred_element_type=jnp.float32)
          + jnp.dot(xin, c0_ref[...], preferred_element_type=jnp.float32)
          + bias0_ref[...])
    y1 = (jnp.dot(xn, a1_ref[...], preferred_element_type=jnp.float32)
          + jnp.dot(xin, c1_ref[...], preferred_element_type=jnp.float32)
          + bias1_ref[...])
    m = m_ref[...]
    a = Z * y1 + (1.0 - Z) * y0
    b = Z * y0 + (1.0 - Z) * y1
    out_ref[...] = b + m * (a - b)


def kernel(x_, edge_index, edge_weight, mask, W_t0, b_t0, W_t1, b_t1,
           W_c0, b_c0, W_c1, b_c1, gamma, beta):
    n, d_in = x_.shape
    e_total = edge_weight.shape[0]
    ei = edge_index.astype(jnp.int32)
    w = edge_weight.astype(jnp.float32)
    m = mask.astype(jnp.float32)
    grid = n // R

    xb = pl.pallas_call(
        _pre_body,
        grid=(grid,),
        in_specs=[
            pl.BlockSpec((R, D), lambda i: (i, 0)),
            pl.BlockSpec((R, 1), lambda i: (i, 0)),
            pl.BlockSpec((D, D), lambda i: (0, 0)),
            pl.BlockSpec((1, D), lambda i: (0, 0)),
            pl.BlockSpec((D, D), lambda i: (0, 0)),
            pl.BlockSpec((1, D), lambda i: (0, 0)),
        ],
        out_specs=pl.BlockSpec((R, D), lambda i: (i, 0)),
        out_shape=jax.ShapeDtypeStruct((n, D), jnp.float32),
    )(x_, m, W_t0.T, b_t0.reshape(1, D), W_t1.T, b_t1.reshape(1, D))

    npad = ((n // NS + 7) // 8 * 8) * NS
    rpt = npad // NS
    epw_raw = e_total // NW
    nsuper = -(-epw_raw // (K * SJ))
    nchunk = nsuper * SJ
    pad = NW * nchunk * K - e_total
    if pad:
        pe = jnp.concatenate([jnp.full((1, pad), n, jnp.int32),
                              jnp.zeros((1, pad), jnp.int32)])
        ei = jnp.concatenate([ei, pe], axis=1)
        w = jnp.concatenate([w, jnp.zeros((pad,), jnp.float32)])
    ei5 = ei.reshape(2, NW, nsuper, SJ, K)
    w5 = w.reshape(NW, nsuper, SJ, K)
    zx = jnp.zeros((rpt, D), jnp.float32)
    zw = jnp.zeros((rpt, 16), jnp.float32)

    epw = nchunk * K
    kd = epw // 5 if (epw % 5 == 0 and (epw // 5) % 16 == 0) else K * SJ
    nchkd = epw // kd
    pw = pl.kernel(
        _deg_body,
        out_type=jax.ShapeDtypeStruct((NC, npad, 16), jnp.float32),
        mesh=plsc.VectorSubcoreMesh(core_axis_name="c", subcore_axis_name="s"),
        compiler_params=pltpu.CompilerParams(use_tc_tiling_on_sc=False),
        scratch_types=[
            pltpu.VMEM((nchkd, kd), jnp.int32),
            pltpu.VMEM((nchkd, kd), jnp.float32),
            pltpu.VMEM((kd, 16), jnp.float32),
            pltpu.VMEM_SHARED((npad, 16), jnp.float32),
        ],
    )(ei.reshape(2, NW, nchkd, kd), w.reshape(NW, nchkd, kd), zw)

    px = pl.kernel(
        _sc_body,
        out_type=jax.ShapeDtypeStruct((NC, npad, D), jnp.float32),
        mesh=plsc.VectorSubcoreMesh(core_axis_name="c", subcore_axis_name="s"),
        scratch_types=[
            pltpu.VMEM((2, SJ, K), jnp.int32),
            pltpu.VMEM((2, SJ, K), jnp.int32),
            pltpu.VMEM((2, SJ, K), jnp.float32),
            pltpu.VMEM((K, D), jnp.float32),
            pltpu.VMEM((K, D), jnp.float32),
            pltpu.VMEM((K, D), jnp.float32),
            pltpu.SemaphoreType.DMA,
            pltpu.SemaphoreType.DMA,
            pltpu.SemaphoreType.DMA,
            pltpu.SemaphoreType.DMA,
            pltpu.SemaphoreType.DMA,
            pltpu.SemaphoreType.DMA,
            pltpu.SemaphoreType.DMA,
            pltpu.VMEM_SHARED((npad, D), jnp.float32),
        ],
    )(ei5, w5, xb, zx)

    out = pl.pallas_call(
        _post_body,
        grid=(grid,),
        in_specs=[
            pl.BlockSpec((NC, R, D), lambda i: (0, i, 0)),
            pl.BlockSpec((NC, R, 16), lambda i: (0, i, 0)),
            pl.BlockSpec((R, D), lambda i: (i, 0)),
            pl.BlockSpec((R, 1), lambda i: (i, 0)),
            pl.BlockSpec((D, D), lambda i: (0, 0)),
            pl.BlockSpec((D, D), lambda i: (0, 0)),
            pl.BlockSpec((D, D), lambda i: (0, 0)),
            pl.BlockSpec((D, D), lambda i: (0, 0)),
            pl.BlockSpec((1, D), lambda i: (0, 0)),
            pl.BlockSpec((1, D), lambda i: (0, 0)),
            pl.BlockSpec((1, D), lambda i: (0, 0)),
            pl.BlockSpec((1, D), lambda i: (0, 0)),
        ],
        out_specs=pl.BlockSpec((R, D), lambda i: (i, 0)),
        out_shape=jax.ShapeDtypeStruct((n, D), jnp.float32),
    )(px, pw, x_, m,
      W_c0[:, :D].T, W_c0[:, D:].T, W_c1[:, :D].T, W_c1[:, D:].T,
      b_c0.reshape(1, D), b_c1.reshape(1, D),
      gamma.reshape(1, D), beta.reshape(1, D))
    return out


# speedup vs baseline: 3.0077x; 3.0077x over previous
import jax
import jax.numpy as jnp
from jax import lax
from jax.experimental import pallas as pl
from jax.experimental.pallas import tpu as pltpu
from jax.experimental.pallas import tpu_sc as plsc

Z = 0.8
NC, NS = 2, 16
NW = NC * NS
K = 80
SJ = 5
R = 400
D = 128


def _pre_body(x_ref, m_ref, w0t_ref, b0_ref, w1t_ref, b1_ref, out_ref):
    x = x_ref[...]
    x0 = jnp.maximum(
        jnp.dot(x, w0t_ref[...], preferred_element_type=jnp.float32)
        + b0_ref[...], 0.0)
    x1 = jnp.maximum(
        jnp.dot(x, w1t_ref[...], preferred_element_type=jnp.float32)
        + b1_ref[...], 0.0)
    m = m_ref[...]
    a = Z * x1 + (1.0 - Z) * x0
    b = Z * x0 + (1.0 - Z) * x1
    out_ref[...] = b + m * (a - b)


def _sc_body(ei5_hbm, w5_hbm, xb_hbm, zx_hbm, px_hbm,
             src_sb, dst_sb, w_sb, r0, r1, r2,
             g0, g1, g2, s0, s1, s2, isem, acc_x):
    rows = (r0, r1, r2)
    gsem = (g0, g1, g2)
    ssem = (s0, s1, s2)
    NB = 3
    npad = acc_x.shape[0]
    nsuper = ei5_hbm.shape[2]
    nchunk = nsuper * SJ
    c = lax.axis_index("c")
    s = lax.axis_index("s")
    wid = c * NS + s
    rpt = npad // NS
    lane0 = jnp.where(lax.iota(jnp.int32, 16) == 0, 1.0, 0.0)
    gdn = lax.GatherDimensionNumbers(
        offset_dims=(), collapsed_slice_dims=(0,), start_index_map=(0,))

    pltpu.sync_copy(zx_hbm, acc_x.at[pl.ds(s * rpt, rpt)])
    plsc.subcore_barrier()

    def super_load_async(p, slot):
        pltpu.async_copy(ei5_hbm.at[0, wid, p], src_sb.at[slot], isem)
        pltpu.async_copy(ei5_hbm.at[1, wid, p], dst_sb.at[slot], isem)
        pltpu.async_copy(w5_hbm.at[wid, p], w_sb.at[slot], isem)

    def super_wait():
        pltpu.make_async_copy(ei5_hbm.at[0, wid, 0], src_sb.at[0], isem).wait()
        pltpu.make_async_copy(ei5_hbm.at[1, wid, 0], dst_sb.at[0], isem).wait()
        pltpu.make_async_copy(w5_hbm.at[wid, 0], w_sb.at[0], isem).wait()

    def scale_chunk(pq, jj, rb):
        @plsc.parallel_loop(0, K // 16, 1, unroll=K // 16)
        def escale(g):
            wg = w_sb[pq, jj, pl.ds(g * 16, 16)]
            for l in range(16):
                idx = (jnp.zeros((16,), jnp.int32) + l).reshape(16, 1)
                wb = lax.gather(wg, idx, gdn, (1,),
                                mode=lax.GatherScatterMode.PROMISE_IN_BOUNDS)
                e = g * 16 + l
                for d in range(D // 16):
                    rb[e, pl.ds(d * 16, 16)] = rb[e, pl.ds(d * 16, 16)] * wb

    def iter_one(j, b, wait_pred, gather_j2, gather_pred=None):
        bp = (b + 2) % NB
        pq = (j // SJ) % 2
        jj = j % SJ
        pltpu.make_async_copy(xb_hbm.at[dst_sb.at[pq, jj]], rows[b],
                              gsem[b]).wait()
        scale_chunk(pq, jj, rows[b])

        def retire():
            pltpu.make_async_copy(rows[bp], acc_x.at[src_sb.at[pq, jj]],
                                  ssem[bp]).wait()
        if wait_pred is None:
            retire()
        else:
            pl.when(wait_pred)(retire)

        if gather_j2 is not None:
            p = j // SJ
            pl.when((jj == 1) & (p + 1 < nsuper))(
                lambda: super_load_async(p + 1, (p + 1) % 2))
            pl.when((jj == SJ - 2) & (gather_j2 < nchunk))(super_wait)

            def launch():
                gpq = (gather_j2 // SJ) % 2
                gjj = gather_j2 % SJ
                pltpu.async_copy(xb_hbm.at[dst_sb.at[gpq, gjj]], rows[bp],
                                 gsem[bp])
            if gather_pred is None:
                launch()
            else:
                pl.when(gather_pred)(launch)
        pltpu.async_copy(rows[b], acc_x.at[src_sb.at[pq, jj]], ssem[b],
                         add=True)

    pltpu.sync_copy(ei5_hbm.at[0, wid, 0], src_sb.at[0])
    pltpu.sync_copy(ei5_hbm.at[1, wid, 0], dst_sb.at[0])
    pltpu.sync_copy(w5_hbm.at[wid, 0], w_sb.at[0])
    super_load_async(1, 1)
    pltpu.async_copy(xb_hbm.at[dst_sb.at[0, 0]], rows[0], gsem[0])
    pltpu.async_copy(xb_hbm.at[dst_sb.at[0, 1]], rows[1], gsem[1])

    nloop = nchunk // 3

    def body(i, carry):
        j0 = 3 * i
        iter_one(j0, 0, i > 0, j0 + 2, j0 + 2 < nchunk)
        iter_one(j0 + 1, 1, None, j0 + 3, j0 + 3 < nchunk)
        iter_one(j0 + 2, 2, None, j0 + 4, j0 + 4 < nchunk)
        return carry
    lax.fori_loop(0, nloop, body, 0)
    for je in range(3 * nloop, nchunk):
        iter_one(je, je % 3, None, None)
    blast = (nchunk - 1) % 3
    lq, lj = ((nchunk - 1) // SJ) % 2, (nchunk - 1) % SJ
    pltpu.make_async_copy(rows[blast], acc_x.at[src_sb.at[lq, lj]],
                          ssem[blast]).wait()

    plsc.subcore_barrier()
    pltpu.sync_copy(acc_x.at[pl.ds(s * rpt, rpt)],
                    px_hbm.at[c, pl.ds(s * rpt, rpt)])


def _deg_body(ei3_hbm, w3_hbm, zw_hbm, pw_hbm, src_a, w_a, wrow_v, acc_w):
    npad = acc_w.shape[0]
    nchkd = src_a.shape[0]
    kd = src_a.shape[1]
    c = lax.axis_index("c")
    s = lax.axis_index("s")
    wid = c * NS + s
    rpt = npad // NS
    lane0 = jnp.where(lax.iota(jnp.int32, 16) == 0, 1.0, 0.0)
    gdn = lax.GatherDimensionNumbers(
        offset_dims=(), collapsed_slice_dims=(0,), start_index_map=(0,))

    pltpu.sync_copy(zw_hbm, acc_w.at[pl.ds(s * rpt, rpt)])
    pltpu.sync_copy(ei3_hbm.at[0, wid], src_a)
    pltpu.sync_copy(w3_hbm.at[wid], w_a)
    plsc.subcore_barrier()

    def chunk(j, carry):
        @plsc.parallel_loop(0, kd // 16, 1, unroll=8)
        def build(g):
            wg = w_a[j, pl.ds(g * 16, 16)]
            for l in range(16):
                idx = (jnp.zeros((16,), jnp.int32) + l).reshape(16, 1)
                wb = lax.gather(wg, idx, gdn, (1,),
                                mode=lax.GatherScatterMode.PROMISE_IN_BOUNDS)
                wrow_v[g * 16 + l, pl.ds(0, 16)] = wb * lane0
        pltpu.sync_copy(wrow_v, acc_w.at[src_a.at[j]], add=True)
        return carry
    lax.fori_loop(0, nchkd, chunk, 0)

    plsc.subcore_barrier()
    pltpu.sync_copy(acc_w.at[pl.ds(s * rpt, rpt)],
                    pw_hbm.at[c, pl.ds(s * rpt, rpt)])


def _post_body(px_ref, pw_ref, x_ref, m_ref,
               a0_ref, c0_ref, a1_ref, c1_ref,
               bias0_ref, bias1_ref, g_ref, be_ref, out_ref):
    accx = px_ref[0] + px_ref[1]
    deg = pw_ref[0, :, 0:1] + pw_ref[1, :, 0:1]
    deg = jnp.where(deg < 0.5, deg + 1.0, deg)
    xm = accx / deg
    mu = jnp.mean(xm, axis=1, keepdims=True)
    var = jnp.mean((xm - mu) * (xm - mu), axis=1, keepdims=True)
    xn = (xm - mu) * lax.rsqrt(var + 1e-5) * g_ref[...] + be_ref[...]
    xin = x_ref[...]
    y0 = (jnp.dot(xn, a0_ref[...], preferred_element_type=jnp.float32)
          + jnp.dot(xin, c0_ref[...], preferred_element_type=jnp.float32)
          + bias0_ref[...])
    y1 = (jnp.dot(xn, a1_ref[...], preferred_element_type=jnp.float32)
          + jnp.dot(xin, c1_ref[...], preferred_element_type=jnp.float32)
          + bias1_ref[...])
    m = m_ref[...]
    a = Z * y1 + (1.0 - Z) * y0
    b = Z * y0 + (1.0 - Z) * y1
    out_ref[...] = b + m * (a - b)


def kernel(x_, edge_index, edge_weight, mask, W_t0, b_t0, W_t1, b_t1,
           W_c0, b_c0, W_c1, b_c1, gamma, beta):
    n, d_in = x_.shape
    e_total = edge_weight.shape[0]
    ei = edge_index.astype(jnp.int32)
    w = edge_weight.astype(jnp.float32)
    m = mask.astype(jnp.float32)
    grid = n // R

    xb = pl.pallas_call(
        _pre_body,
        grid=(grid,),
        in_specs=[
            pl.BlockSpec((R, D), lambda i: (i, 0)),
            pl.BlockSpec((R, 1), lambda i: (i, 0)),
            pl.BlockSpec((D, D), lambda i: (0, 0)),
            pl.BlockSpec((1, D), lambda i: (0, 0)),
            pl.BlockSpec((D, D), lambda i: (0, 0)),
            pl.BlockSpec((1, D), lambda i: (0, 0)),
        ],
        out_specs=pl.BlockSpec((R, D), lambda i: (i, 0)),
        out_shape=jax.ShapeDtypeStruct((n, D), jnp.float32),
    )(x_, m, W_t0.T, b_t0.reshape(1, D), W_t1.T, b_t1.reshape(1, D))

    npad = ((n // NS + 7) // 8 * 8) * NS
    rpt = npad // NS
    epw_raw = e_total // NW
    nsuper = -(-epw_raw // (K * SJ))
    nchunk = nsuper * SJ
    pad = NW * nchunk * K - e_total
    if pad:
        pe = jnp.concatenate([jnp.full((1, pad), n, jnp.int32),
                              jnp.zeros((1, pad), jnp.int32)])
        ei = jnp.concatenate([ei, pe], axis=1)
        w = jnp.concatenate([w, jnp.zeros((pad,), jnp.float32)])
    ei5 = ei.reshape(2, NW, nsuper, SJ, K)
    w5 = w.reshape(NW, nsuper, SJ, K)
    zx = jnp.zeros((rpt, D), jnp.float32)
    zw = jnp.zeros((rpt, 16), jnp.float32)

    epw = nchunk * K
    kd = epw // 5 if (epw % 5 == 0 and (epw // 5) % 16 == 0) else K * SJ
    nchkd = epw // kd
    pw = pl.kernel(
        _deg_body,
        out_type=jax.ShapeDtypeStruct((NC, npad, 16), jnp.float32),
        mesh=plsc.VectorSubcoreMesh(core_axis_name="c", subcore_axis_name="s"),
        compiler_params=pltpu.CompilerParams(use_tc_tiling_on_sc=False),
        scratch_types=[
            pltpu.VMEM((nchkd, kd), jnp.int32),
            pltpu.VMEM((nchkd, kd), jnp.float32),
            pltpu.VMEM((kd, 16), jnp.float32),
            pltpu.VMEM_SHARED((npad, 16), jnp.float32),
        ],
    )(ei.reshape(2, NW, nchkd, kd), w.reshape(NW, nchkd, kd), zw)

    px = pl.kernel(
        _sc_body,
        out_type=jax.ShapeDtypeStruct((NC, npad, D), jnp.float32),
        mesh=plsc.VectorSubcoreMesh(core_axis_name="c", subcore_axis_name="s"),
        scratch_types=[
            pltpu.VMEM((2, SJ, K), jnp.int32),
            pltpu.VMEM((2, SJ, K), jnp.int32),
            pltpu.VMEM((2, SJ, K), jnp.float32),
            pltpu.VMEM((K, D), jnp.float32),
            pltpu.VMEM((K, D), jnp.float32),
            pltpu.VMEM((K, D), jnp.float32),
            pltpu.SemaphoreType.DMA,
            pltpu.SemaphoreType.DMA,
            pltpu.SemaphoreType.DMA,
            pltpu.SemaphoreType.DMA,
            pltpu.SemaphoreType.DMA,
            pltpu.SemaphoreType.DMA,
            pltpu.SemaphoreType.DMA,
            pltpu.VMEM_SHARED((npad, D), jnp.float32),
        ],
    )(ei5, w5, xb, zx)

    out = pl.pallas_call(
        _post_body,
        grid=(grid,),
        in_specs=[
            pl.BlockSpec((NC, R, D), lambda i: (0, i, 0)),
            pl.BlockSpec((NC, R, 16), lambda i: (0, i, 0)),
            pl.BlockSpec((R, D), lambda i: (i, 0)),
            pl.BlockSpec((R, 1), lambda i: (i, 0)),
            pl.BlockSpec((D, D), lambda i: (0, 0)),
            pl.BlockSpec((D, D), lambda i: (0, 0)),
            pl.BlockSpec((D, D), lambda i: (0, 0)),
            pl.BlockSpec((D, D), lambda i: (0, 0)),
            pl.BlockSpec((1, D), lambda i: (0, 0)),
            pl.BlockSpec((1, D), lambda i: (0, 0)),
            pl.BlockSpec((1, D), lambda i: (0, 0)),
            pl.BlockSpec((1, D), lambda i: (0, 0)),
        ],
        out_specs=pl.BlockSpec((R, D), lambda i: (i, 0)),
        out_shape=jax.ShapeDtypeStruct((n, D), jnp.float32),
    )(px, pw, x_, m,
      W_c0[:, :D].T, W_c0[:, D:].T, W_c1[:, :D].T, W_c1[:, D:].T,
      b_c0.reshape(1, D), b_c1.reshape(1, D),
      gamma.reshape(1, D), beta.reshape(1, D))
    return out


# speedup vs baseline: 3.0733x; 1.0218x over previous
import jax
import jax.numpy as jnp
from jax import lax
from jax.experimental import pallas as pl
from jax.experimental.pallas import tpu as pltpu
from jax.experimental.pallas import tpu_sc as plsc

Z = 0.8
NC, NS = 2, 16
NW = NC * NS
K = 80
SJ = 5
R = 1000
D = 128


def _pre_body(x_ref, m_ref, w0t_ref, b0_ref, w1t_ref, b1_ref, out_ref):
    x = x_ref[...]
    x0 = jnp.maximum(
        jnp.dot(x, w0t_ref[...], preferred_element_type=jnp.float32)
        + b0_ref[...], 0.0)
    x1 = jnp.maximum(
        jnp.dot(x, w1t_ref[...], preferred_element_type=jnp.float32)
        + b1_ref[...], 0.0)
    m = m_ref[...]
    a = Z * x1 + (1.0 - Z) * x0
    b = Z * x0 + (1.0 - Z) * x1
    out_ref[...] = b + m * (a - b)


def _sc_body(ei5_hbm, w5_hbm, xb_hbm, zx_hbm, px_hbm,
             src_sb, dst_sb, w_sb, r0, r1, r2,
             g0, g1, g2, s0, s1, s2, isem, acc_x):
    rows = (r0, r1, r2)
    gsem = (g0, g1, g2)
    ssem = (s0, s1, s2)
    NB = 3
    npad = acc_x.shape[0]
    nsuper = ei5_hbm.shape[2]
    nchunk = nsuper * SJ
    c = lax.axis_index("c")
    s = lax.axis_index("s")
    wid = c * NS + s
    rpt = npad // NS
    lane0 = jnp.where(lax.iota(jnp.int32, 16) == 0, 1.0, 0.0)
    gdn = lax.GatherDimensionNumbers(
        offset_dims=(), collapsed_slice_dims=(0,), start_index_map=(0,))

    pltpu.sync_copy(zx_hbm, acc_x.at[pl.ds(s * rpt, rpt)])
    plsc.subcore_barrier()

    def super_load_async(p, slot):
        pltpu.async_copy(ei5_hbm.at[0, wid, p], src_sb.at[slot], isem)
        pltpu.async_copy(ei5_hbm.at[1, wid, p], dst_sb.at[slot], isem)
        pltpu.async_copy(w5_hbm.at[wid, p], w_sb.at[slot], isem)

    def super_wait():
        pltpu.make_async_copy(ei5_hbm.at[0, wid, 0], src_sb.at[0], isem).wait()
        pltpu.make_async_copy(ei5_hbm.at[1, wid, 0], dst_sb.at[0], isem).wait()
        pltpu.make_async_copy(w5_hbm.at[wid, 0], w_sb.at[0], isem).wait()

    def scale_chunk(pq, jj, rb):
        @plsc.parallel_loop(0, K // 16, 1, unroll=K // 16)
        def escale(g):
            wg = w_sb[pq, jj, pl.ds(g * 16, 16)]
            for l in range(16):
                idx = (jnp.zeros((16,), jnp.int32) + l).reshape(16, 1)
                wb = lax.gather(wg, idx, gdn, (1,),
                                mode=lax.GatherScatterMode.PROMISE_IN_BOUNDS)
                e = g * 16 + l
                for d in range(D // 16):
                    rb[e, pl.ds(d * 16, 16)] = rb[e, pl.ds(d * 16, 16)] * wb

    def iter_one(j, b, wait_pred, gather_j2, gather_pred=None):
        bp = (b + 2) % NB
        pq = (j // SJ) % 2
        jj = j % SJ
        pltpu.make_async_copy(xb_hbm.at[dst_sb.at[pq, jj]], rows[b],
                              gsem[b]).wait()
        scale_chunk(pq, jj, rows[b])

        def retire():
            pltpu.make_async_copy(rows[bp], acc_x.at[src_sb.at[pq, jj]],
                                  ssem[bp]).wait()
        if wait_pred is None:
            retire()
        else:
            pl.when(wait_pred)(retire)

        if gather_j2 is not None:
            p = j // SJ
            pl.when((jj == 1) & (p + 1 < nsuper))(
                lambda: super_load_async(p + 1, (p + 1) % 2))
            pl.when((jj == SJ - 2) & (gather_j2 < nchunk))(super_wait)

            def launch():
                gpq = (gather_j2 // SJ) % 2
                gjj = gather_j2 % SJ
                pltpu.async_copy(xb_hbm.at[dst_sb.at[gpq, gjj]], rows[bp],
                                 gsem[bp])
            if gather_pred is None:
                launch()
            else:
                pl.when(gather_pred)(launch)
        pltpu.async_copy(rows[b], acc_x.at[src_sb.at[pq, jj]], ssem[b],
                         add=True)

    pltpu.sync_copy(ei5_hbm.at[0, wid, 0], src_sb.at[0])
    pltpu.sync_copy(ei5_hbm.at[1, wid, 0], dst_sb.at[0])
    pltpu.sync_copy(w5_hbm.at[wid, 0], w_sb.at[0])
    super_load_async(1, 1)
    pltpu.async_copy(xb_hbm.at[dst_sb.at[0, 0]], rows[0], gsem[0])
    pltpu.async_copy(xb_hbm.at[dst_sb.at[0, 1]], rows[1], gsem[1])

    nloop = nchunk // 3

    def body(i, carry):
        j0 = 3 * i
        iter_one(j0, 0, i > 0, j0 + 2, j0 + 2 < nchunk)
        iter_one(j0 + 1, 1, None, j0 + 3, j0 + 3 < nchunk)
        iter_one(j0 + 2, 2, None, j0 + 4, j0 + 4 < nchunk)
        return carry
    lax.fori_loop(0, nloop, body, 0)
    for je in range(3 * nloop, nchunk):
        iter_one(je, je % 3, None, None)
    blast = (nchunk - 1) % 3
    lq, lj = ((nchunk - 1) // SJ) % 2, (nchunk - 1) % SJ
    pltpu.make_async_copy(rows[blast], acc_x.at[src_sb.at[lq, lj]],
                          ssem[blast]).wait()

    plsc.subcore_barrier()
    pltpu.sync_copy(acc_x.at[pl.ds(s * rpt, rpt)],
                    px_hbm.at[c, pl.ds(s * rpt, rpt)])


def _deg_body(ei3_hbm, w3_hbm, zw_hbm, pw_hbm, src_a, w_a, wrow_a, wrow_b,
              wsem_a, wsem_b, acc_w):
    npad = acc_w.shape[0]
    nchkd = src_a.shape[0]
    kd = src_a.shape[1]
    c = lax.axis_index("c")
    s = lax.axis_index("s")
    wid = c * NS + s
    rpt = npad // NS
    lane0 = jnp.where(lax.iota(jnp.int32, 16) == 0, 1.0, 0.0)
    gdn = lax.GatherDimensionNumbers(
        offset_dims=(), collapsed_slice_dims=(0,), start_index_map=(0,))

    pltpu.sync_copy(zw_hbm, acc_w.at[pl.ds(s * rpt, rpt)])
    pltpu.sync_copy(ei3_hbm.at[0, wid], src_a)
    pltpu.sync_copy(w3_hbm.at[wid], w_a)
    plsc.subcore_barrier()

    wrow = (wrow_a, wrow_b)
    wsem = (wsem_a, wsem_b)
    for j in range(nchkd):
        b = j % 2
        if j >= 2:
            pltpu.make_async_copy(wrow[b], acc_w.at[src_a.at[j - 2]],
                                  wsem[b]).wait()

        @plsc.parallel_loop(0, kd // 16, 1, unroll=8)
        def build(g, _b=b, _j=j):
            wg = w_a[_j, pl.ds(g * 16, 16)]
            for l in range(16):
                idx = (jnp.zeros((16,), jnp.int32) + l).reshape(16, 1)
                wb = lax.gather(wg, idx, gdn, (1,),
                                mode=lax.GatherScatterMode.PROMISE_IN_BOUNDS)
                wrow[_b][g * 16 + l, pl.ds(0, 16)] = wb * lane0
        pltpu.async_copy(wrow[b], acc_w.at[src_a.at[j]], wsem[b], add=True)
    for j in range(max(0, nchkd - 2), nchkd):
        pltpu.make_async_copy(wrow[j % 2], acc_w.at[src_a.at[j]],
                              wsem[j % 2]).wait()

    plsc.subcore_barrier()
    pltpu.sync_copy(acc_w.at[pl.ds(s * rpt, rpt)],
                    pw_hbm.at[c, pl.ds(s * rpt, rpt)])


def _post_body(px_ref, pw_ref, x_ref, m_ref,
               a0_ref, c0_ref, a1_ref, c1_ref,
               bias0_ref, bias1_ref, g_ref, be_ref, out_ref):
    accx = px_ref[0] + px_ref[1]
    deg = pw_ref[0, :, 0:1] + pw_ref[1, :, 0:1]
    deg = jnp.where(deg < 0.5, deg + 1.0, deg)
    xm = accx / deg
    mu = jnp.mean(xm, axis=1, keepdims=True)
    var = jnp.mean((xm - mu) * (xm - mu), axis=1, keepdims=True)
    xn = (xm - mu) * lax.rsqrt(var + 1e-5) * g_ref[...] + be_ref[...]
    xin = x_ref[...]
    y0 = (jnp.dot(xn, a0_ref[...], preferred_element_type=jnp.float32)
          + jnp.dot(xin, c0_ref[...], preferred_element_type=jnp.float32)
          + bias0_ref[...])
    y1 = (jnp.dot(xn, a1_ref[...], preferred_element_type=jnp.float32)
          + jnp.dot(xin, c1_ref[...], preferred_element_type=jnp.float32)
          + bias1_ref[...])
    m = m_ref[...]
    a = Z * y1 + (1.0 - Z) * y0
    b = Z * y0 + (1.0 - Z) * y1
    out_ref[...] = b + m * (a - b)


def kernel(x_, edge_index, edge_weight, mask, W_t0, b_t0, W_t1, b_t1,
           W_c0, b_c0, W_c1, b_c1, gamma, beta):
    n, d_in = x_.shape
    e_total = edge_weight.shape[0]
    ei = edge_index.astype(jnp.int32)
    w = edge_weight.astype(jnp.float32)
    m = mask.astype(jnp.float32)
    grid = n // R

    xb = pl.pallas_call(
        _pre_body,
        grid=(grid,),
        in_specs=[
            pl.BlockSpec((R, D), lambda i: (i, 0)),
            pl.BlockSpec((R, 1), lambda i: (i, 0)),
            pl.BlockSpec((D, D), lambda i: (0, 0)),
            pl.BlockSpec((1, D), lambda i: (0, 0)),
            pl.BlockSpec((D, D), lambda i: (0, 0)),
            pl.BlockSpec((1, D), lambda i: (0, 0)),
        ],
        out_specs=pl.BlockSpec((R, D), lambda i: (i, 0)),
        out_shape=jax.ShapeDtypeStruct((n, D), jnp.float32),
    )(x_, m, W_t0.T, b_t0.reshape(1, D), W_t1.T, b_t1.reshape(1, D))

    npad = ((n // NS + 7) // 8 * 8) * NS
    rpt = npad // NS
    epw_raw = e_total // NW
    nsuper = -(-epw_raw // (K * SJ))
    nchunk = nsuper * SJ
    pad = NW * nchunk * K - e_total
    if pad:
        pe = jnp.concatenate([jnp.full((1, pad), n, jnp.int32),
                              jnp.zeros((1, pad), jnp.int32)])
        ei = jnp.concatenate([ei, pe], axis=1)
        w = jnp.concatenate([w, jnp.zeros((pad,), jnp.float32)])
    ei5 = ei.reshape(2, NW, nsuper, SJ, K)
    w5 = w.reshape(NW, nsuper, SJ, K)
    zx = jnp.zeros((rpt, D), jnp.float32)
    zw = jnp.zeros((rpt, 16), jnp.float32)

    epw = nchunk * K
    kd = epw // 5 if (epw % 5 == 0 and (epw // 5) % 16 == 0) else K * SJ
    nchkd = epw // kd
    pw = pl.kernel(
        _deg_body,
        out_type=jax.ShapeDtypeStruct((NC, npad, 16), jnp.float32),
        mesh=plsc.VectorSubcoreMesh(core_axis_name="c", subcore_axis_name="s"),
        compiler_params=pltpu.CompilerParams(use_tc_tiling_on_sc=False),
        scratch_types=[
            pltpu.VMEM((nchkd, kd), jnp.int32),
            pltpu.VMEM((nchkd, kd), jnp.float32),
            pltpu.VMEM((kd, 16), jnp.float32),
            pltpu.VMEM((kd, 16), jnp.float32),
            pltpu.SemaphoreType.DMA,
            pltpu.SemaphoreType.DMA,
            pltpu.VMEM_SHARED((npad, 16), jnp.float32),
        ],
    )(ei.reshape(2, NW, nchkd, kd), w.reshape(NW, nchkd, kd), zw)

    px = pl.kernel(
        _sc_body,
        out_type=jax.ShapeDtypeStruct((NC, npad, D), jnp.float32),
        mesh=plsc.VectorSubcoreMesh(core_axis_name="c", subcore_axis_name="s"),
        scratch_types=[
            pltpu.VMEM((2, SJ, K), jnp.int32),
            pltpu.VMEM((2, SJ, K), jnp.int32),
            pltpu.VMEM((2, SJ, K), jnp.float32),
            pltpu.VMEM((K, D), jnp.float32),
            pltpu.VMEM((K, D), jnp.float32),
            pltpu.VMEM((K, D), jnp.float32),
            pltpu.SemaphoreType.DMA,
            pltpu.SemaphoreType.DMA,
            pltpu.SemaphoreType.DMA,
            pltpu.SemaphoreType.DMA,
            pltpu.SemaphoreType.DMA,
            pltpu.SemaphoreType.DMA,
            pltpu.SemaphoreType.DMA,
            pltpu.VMEM_SHARED((npad, D), jnp.float32),
        ],
    )(ei5, w5, xb, zx)

    out = pl.pallas_call(
        _post_body,
        grid=(grid,),
        in_specs=[
            pl.BlockSpec((NC, R, D), lambda i: (0, i, 0)),
            pl.BlockSpec((NC, R, 16), lambda i: (0, i, 0)),
            pl.BlockSpec((R, D), lambda i: (i, 0)),
            pl.BlockSpec((R, 1), lambda i: (i, 0)),
            pl.BlockSpec((D, D), lambda i: (0, 0)),
            pl.BlockSpec((D, D), lambda i: (0, 0)),
            pl.BlockSpec((D, D), lambda i: (0, 0)),
            pl.BlockSpec((D, D), lambda i: (0, 0)),
            pl.BlockSpec((1, D), lambda i: (0, 0)),
            pl.BlockSpec((1, D), lambda i: (0, 0)),
            pl.BlockSpec((1, D), lambda i: (0, 0)),
            pl.BlockSpec((1, D), lambda i: (0, 0)),
        ],
        out_specs=pl.BlockSpec((R, D), lambda i: (i, 0)),
        out_shape=jax.ShapeDtypeStruct((n, D), jnp.float32),
    )(px, pw, x_, m,
      W_c0[:, :D].T, W_c0[:, D:].T, W_c1[:, :D].T, W_c1[:, D:].T,
      b_c0.reshape(1, D), b_c1.reshape(1, D),
      gamma.reshape(1, D), beta.reshape(1, D))
    return out
